# baseline (speedup 1.0000x reference)
import functools

import jax
import jax.numpy as jnp
from jax import lax
from jax.experimental import pallas as pl
from jax.experimental.pallas import tpu as pltpu
from jax.experimental.pallas import tpu_sc as plsc

NC = 2
NS = 16
NW = NC * NS
L = 16
BATCH = 16384
K = 16
BPW = BATCH // NW
CHUNK = 128
NCH = BPW // CHUNK
NBLK = CHUNK // L

NU = 100000
NV = 1000000


def _mf_body(uidx_hbm, vidx_hbm, wf_hbm, hf_hbm, out_hbm,
             idsu_v, idsv_v, ubuf_v, vbuf_v, out_v, sems):
    wid = lax.axis_index("s") * NC + lax.axis_index("c")
    base = wid * BPW

    pltpu.sync_copy(uidx_hbm.at[pl.ds(base, BPW)], idsu_v)
    pltpu.sync_copy(vidx_hbm.at[pl.ds(base, BPW)], idsv_v)

    handles = []
    for j in range(NCH):
        hs = []
        for k in range(K):
            hs.append(pltpu.async_copy(
                wf_hbm.at[pl.ds(k * NU, NU)].at[idsu_v.at[pl.ds(j * CHUNK, CHUNK)]],
                ubuf_v.at[j, k], sems.at[j]))
            hs.append(pltpu.async_copy(
                hf_hbm.at[pl.ds(k * NV, NV)].at[idsv_v.at[pl.ds(j * CHUNK, CHUNK)]],
                vbuf_v.at[j, k], sems.at[j]))
        handles.append(hs)

    for j in range(NCH):
        for h in handles[j]:
            h.wait()

        def blk_body(b, carry, j=j):
            acc = jnp.zeros((L,), jnp.float32)
            for k in range(K):
                u = ubuf_v[j, k, pl.ds(b * L, L)]
                v = vbuf_v[j, k, pl.ds(b * L, L)]
                acc = acc + u * v
            out_v[pl.ds(j * CHUNK + b * L, L)] = acc
            return carry

        lax.fori_loop(0, NBLK, blk_body, 0)

    pltpu.sync_copy(out_v, out_hbm.at[pl.ds(base, BPW)])


@jax.jit
def kernel(x, W, H):
    u_idx = x[:, 0].astype(jnp.int32)
    v_idx = x[:, 1].astype(jnp.int32)
    w_flat = W.T.reshape(-1)
    h_flat = H.T.reshape(-1)
    mf = functools.partial(
        pl.kernel,
        mesh=plsc.VectorSubcoreMesh(core_axis_name="c", subcore_axis_name="s"),
        out_type=jax.ShapeDtypeStruct((BATCH,), jnp.float32),
        compiler_params=pltpu.CompilerParams(
            needs_layout_passes=False,
            use_tc_tiling_on_sc=False,
            disable_bounds_checks=True,
        ),
        scratch_types=[
            pltpu.VMEM((BPW,), jnp.int32),
            pltpu.VMEM((BPW,), jnp.int32),
            pltpu.VMEM((NCH, K, CHUNK), jnp.float32),
            pltpu.VMEM((NCH, K, CHUNK), jnp.float32),
            pltpu.VMEM((BPW,), jnp.float32),
            pltpu.SemaphoreType.DMA((NCH,)),
        ],
    )(_mf_body)
    return mf(u_idx, v_idx, w_flat, h_flat)


# speedup vs baseline: 1.0001x; 1.0001x over previous
import functools

import jax
import jax.numpy as jnp
from jax import lax
from jax.experimental import pallas as pl
from jax.experimental.pallas import tpu as pltpu
from jax.experimental.pallas import tpu_sc as plsc

NC = 2
NS = 16
NW = NC * NS
L = 16
BATCH = 16384
K = 16
BPW = BATCH // NW
CHUNK = 128
NCH = BPW // CHUNK
NBLK = CHUNK // L

NU = 100000
NV = 1000000


def _mf_body(uidx_hbm, vidx_hbm, wf_hbm, hf_hbm, out_hbm,
             idsu_v, idsv_v, ubuf_v, vbuf_v, out_v, sems):
    wid = lax.axis_index("s") * NC + lax.axis_index("c")
    base = wid * BPW

    pltpu.sync_copy(uidx_hbm.at[pl.ds(base, BPW)], idsu_v)
    pltpu.sync_copy(vidx_hbm.at[pl.ds(base, BPW)], idsv_v)

    handles = []
    for j in range(NCH):
        hs = []
        for k in range(K):
            hs.append(pltpu.async_copy(
                wf_hbm.at[k].at[idsu_v.at[pl.ds(j * CHUNK, CHUNK)]],
                ubuf_v.at[j, k], sems.at[j]))
            hs.append(pltpu.async_copy(
                hf_hbm.at[k].at[idsv_v.at[pl.ds(j * CHUNK, CHUNK)]],
                vbuf_v.at[j, k], sems.at[j]))
        handles.append(hs)

    for j in range(NCH):
        for h in handles[j]:
            h.wait()

        def blk_body(b, carry, j=j):
            acc = jnp.zeros((L,), jnp.float32)
            for k in range(K):
                u = ubuf_v[j, k, pl.ds(b * L, L)]
                v = vbuf_v[j, k, pl.ds(b * L, L)]
                acc = acc + u * v
            out_v[pl.ds(j * CHUNK + b * L, L)] = acc
            return carry

        lax.fori_loop(0, NBLK, blk_body, 0)

    pltpu.sync_copy(out_v, out_hbm.at[pl.ds(base, BPW)])


@jax.jit
def kernel(x, W, H):
    u_idx = x[:, 0].astype(jnp.int32)
    v_idx = x[:, 1].astype(jnp.int32)
    w_flat = W.T
    h_flat = H.T
    mf = functools.partial(
        pl.kernel,
        mesh=plsc.VectorSubcoreMesh(core_axis_name="c", subcore_axis_name="s"),
        out_type=jax.ShapeDtypeStruct((BATCH,), jnp.float32),
        compiler_params=pltpu.CompilerParams(
            needs_layout_passes=False,
            use_tc_tiling_on_sc=False,
            disable_bounds_checks=True,
        ),
        scratch_types=[
            pltpu.VMEM((BPW,), jnp.int32),
            pltpu.VMEM((BPW,), jnp.int32),
            pltpu.VMEM((NCH, K, CHUNK), jnp.float32),
            pltpu.VMEM((NCH, K, CHUNK), jnp.float32),
            pltpu.VMEM((BPW,), jnp.float32),
            pltpu.SemaphoreType.DMA((NCH,)),
        ],
    )(_mf_body)
    return mf(u_idx, v_idx, w_flat, h_flat)


# speedup vs baseline: 10.0914x; 10.0908x over previous
import functools

import jax
import jax.numpy as jnp
from jax import lax
from jax.experimental import pallas as pl
from jax.experimental.pallas import tpu as pltpu
from jax.experimental.pallas import tpu_sc as plsc

NC = 2
NS = 16
NW = NC * NS
L = 16
BATCH = 16384
K = 16
BPW = BATCH // NW
CHUNK = 128
NCH = BPW // CHUNK
NBLK = CHUNK // L

NU = 100000
NV = 1000000
CW = (NU + 127) // 128
CH = (NV + 127) // 128
PW = CW * 128
PH = CH * 128

FULL_W = NU // 128
FULL_H = NV // 128
TAIL_W = NU - FULL_W * 128
TAIL_H = NV - FULL_H * 128

PCOLS = 16
PIDS = PCOLS * 128
NP_H = (FULL_H + PCOLS - 1) // PCOLS
NP_W = (FULL_W + PCOLS - 1) // PCOLS
ITER_H = (NP_H + NW - 1) // NW
ITER_W = (NP_W + NW - 1) // NW


def _detile_one(table_hbm, flat_hbm, buf_v, rsems, wsems,
                wid, n_pieces, n_full_cols, stride, n_iter):
    last_start = (n_full_cols - PCOLS) * 128

    for i in range(n_iter):
        p = wid + i * NW

        @pl.when(p < n_pieces)
        def _piece(p=p):
            start = pl.multiple_of(jnp.minimum(p * PIDS, last_start), 128)
            pltpu.sync_copy(table_hbm.at[:, pl.ds(start, PIDS)], buf_v.at[0])

            def write_k(k, carry):
                pltpu.sync_copy(
                    buf_v.at[0, k],
                    flat_hbm.at[pl.ds(k * stride + start, PIDS)])
                return carry

            lax.fori_loop(0, K, write_k, 0)


def _fmt_body(wt_hbm, ht_hbm, twt_hbm, tht_hbm, wf_hbm, hf_hbm,
              buf_v, tw_v, th_v, rsems, wsems):
    wid = lax.axis_index("s") * NC + lax.axis_index("c")

    @pl.when(wid == 0)
    def _tails():
        pltpu.sync_copy(twt_hbm, tw_v)
        pltpu.sync_copy(tht_hbm, th_v)
        for k in range(K):
            pltpu.sync_copy(
                tw_v.at[k], wf_hbm.at[pl.ds(k * PW + FULL_W * 128, TAIL_W)])
            pltpu.sync_copy(
                th_v.at[k], hf_hbm.at[pl.ds(k * PH + FULL_H * 128, TAIL_H)])

    _detile_one(wt_hbm, wf_hbm, buf_v, rsems, wsems,
                wid, NP_W, FULL_W, PW, ITER_W)
    _detile_one(ht_hbm, hf_hbm, buf_v, rsems, wsems,
                wid, NP_H, FULL_H, PH, ITER_H)


def _mf_body(uidx_hbm, vidx_hbm, wf_hbm, hf_hbm, out_hbm,
             idsu_v, idsv_v, ubuf_v, vbuf_v, out_v, sems):
    wid = lax.axis_index("s") * NC + lax.axis_index("c")
    base = wid * BPW

    pltpu.sync_copy(uidx_hbm.at[pl.ds(base, BPW)], idsu_v)
    pltpu.sync_copy(vidx_hbm.at[pl.ds(base, BPW)], idsv_v)

    handles = []
    for j in range(NCH):
        hs = []
        for k in range(K):
            hs.append(pltpu.async_copy(
                wf_hbm.at[pl.ds(k * PW, PW)]
                .at[idsu_v.at[pl.ds(j * CHUNK, CHUNK)]],
                ubuf_v.at[j, k], sems.at[j]))
            hs.append(pltpu.async_copy(
                hf_hbm.at[pl.ds(k * PH, PH)]
                .at[idsv_v.at[pl.ds(j * CHUNK, CHUNK)]],
                vbuf_v.at[j, k], sems.at[j]))
        handles.append(hs)

    for j in range(NCH):
        for h in handles[j]:
            h.wait()

        def blk_body(b, carry, j=j):
            acc = jnp.zeros((L,), jnp.float32)
            for k in range(K):
                u = ubuf_v[j, k, pl.ds(b * L, L)]
                v = vbuf_v[j, k, pl.ds(b * L, L)]
                acc = acc + u * v
            out_v[pl.ds(j * CHUNK + b * L, L)] = acc
            return carry

        lax.fori_loop(0, NBLK, blk_body, 0)

    pltpu.sync_copy(out_v, out_hbm.at[pl.ds(base, BPW)])


_MESH = dict(core_axis_name="c", subcore_axis_name="s")


@jax.jit
def kernel(x, W, H):
    u_idx = x[:, 0].astype(jnp.int32)
    v_idx = x[:, 1].astype(jnp.int32)

    fmt = functools.partial(
        pl.kernel,
        mesh=plsc.VectorSubcoreMesh(**_MESH),
        out_type=(
            jax.ShapeDtypeStruct((K * PW,), jnp.float32),
            jax.ShapeDtypeStruct((K * PH,), jnp.float32),
        ),
        compiler_params=pltpu.CompilerParams(
            needs_layout_passes=False,
            use_tc_tiling_on_sc=True,
            disable_bounds_checks=True,
        ),
        scratch_types=[
            pltpu.VMEM((2, K, PIDS), jnp.float32),
            pltpu.VMEM((K, TAIL_W), jnp.float32),
            pltpu.VMEM((K, TAIL_H), jnp.float32),
            pltpu.SemaphoreType.DMA((2,)),
            pltpu.SemaphoreType.DMA((2,)),
        ],
    )(_fmt_body)
    wt = W.T
    ht = H.T
    w_flat, h_flat = fmt(wt, ht, wt[:, FULL_W * 128:], ht[:, FULL_H * 128:])

    mf = functools.partial(
        pl.kernel,
        mesh=plsc.VectorSubcoreMesh(**_MESH),
        out_type=jax.ShapeDtypeStruct((BATCH,), jnp.float32),
        compiler_params=pltpu.CompilerParams(
            needs_layout_passes=False,
            use_tc_tiling_on_sc=False,
            disable_bounds_checks=True,
        ),
        scratch_types=[
            pltpu.VMEM((BPW,), jnp.int32),
            pltpu.VMEM((BPW,), jnp.int32),
            pltpu.VMEM((NCH, K, CHUNK), jnp.float32),
            pltpu.VMEM((NCH, K, CHUNK), jnp.float32),
            pltpu.VMEM((BPW,), jnp.float32),
            pltpu.SemaphoreType.DMA((NCH,)),
        ],
    )(_mf_body)
    return mf(u_idx, v_idx, w_flat, h_flat)


# speedup vs baseline: 11.0244x; 1.0925x over previous
import functools

import jax
import jax.numpy as jnp
from jax import lax
from jax.experimental import pallas as pl
from jax.experimental.pallas import tpu as pltpu
from jax.experimental.pallas import tpu_sc as plsc

NC = 2
NS = 16
NW = NC * NS
L = 16
BATCH = 16384
K = 16
BPW = BATCH // NW
CHUNK = 128
NCH = BPW // CHUNK
NBLK = CHUNK // L

NU = 100000
NV = 1000000
CW = (NU + 127) // 128
CH = (NV + 127) // 128
PW = CW * 128
PH = CH * 128

FULL_W = NU // 128
FULL_H = NV // 128
TAIL_W = NU - FULL_W * 128
TAIL_H = NV - FULL_H * 128

PCOLS = 24
PIDS = PCOLS * 128
NP_H = (FULL_H + PCOLS - 1) // PCOLS
NP_W = (FULL_W + PCOLS - 1) // PCOLS
ITER_H = (NP_H + NW - 1) // NW
ITER_W = (NP_W + NW - 1) // NW


def _detile_one(table_hbm, flat_hbm, buf_v, wsems,
                wid, n_pieces, n_full_cols, stride, n_iter):
    last_start = (n_full_cols - PCOLS) * 128

    def piece_start(i):
        return pl.multiple_of(
            jnp.minimum((wid + i * NW) * PIDS, last_start), 128)

    def write_row(b, k, start):
        return pltpu.async_copy(
            buf_v.at[b, k],
            flat_hbm.at[pl.ds(pl.multiple_of(k * stride + start, 128), PIDS)],
            wsems.at[b])

    n_base = n_pieces // NW
    handles = [None] * n_base
    for i in range(n_base):
        b = i % 2
        if i >= 2:
            for h in handles[i - 2]:
                h.wait()
        start = piece_start(i)
        pltpu.sync_copy(table_hbm.at[:, pl.ds(start, PIDS)], buf_v.at[b])
        handles[i] = [write_row(b, k, start) for k in range(K)]
    for i in range(max(n_base - 2, 0), n_base):
        for h in handles[i]:
            h.wait()

    if n_pieces % NW:

        @pl.when(wid + n_base * NW < n_pieces)
        def _tail_piece():
            start = piece_start(n_base)
            pltpu.sync_copy(table_hbm.at[:, pl.ds(start, PIDS)], buf_v.at[0])
            for k in range(K):
                pltpu.sync_copy(
                    buf_v.at[0, k],
                    flat_hbm.at[pl.ds(
                        pl.multiple_of(k * stride + start, 128), PIDS)])


def _fmt_body(wt_hbm, ht_hbm, twt_hbm, tht_hbm, wf_hbm, hf_hbm,
              buf_v, tw_v, th_v, wsems):
    wid = lax.axis_index("s") * NC + lax.axis_index("c")

    @pl.when(wid == 0)
    def _tails():
        pltpu.sync_copy(twt_hbm, tw_v)
        pltpu.sync_copy(tht_hbm, th_v)
        for k in range(K):
            pltpu.sync_copy(
                tw_v.at[k], wf_hbm.at[pl.ds(k * PW + FULL_W * 128, TAIL_W)])
            pltpu.sync_copy(
                th_v.at[k], hf_hbm.at[pl.ds(k * PH + FULL_H * 128, TAIL_H)])

    _detile_one(wt_hbm, wf_hbm, buf_v, wsems,
                wid, NP_W, FULL_W, PW, ITER_W)
    _detile_one(ht_hbm, hf_hbm, buf_v, wsems,
                wid, NP_H, FULL_H, PH, ITER_H)


def _mf_body(uidx_hbm, vidx_hbm, wf_hbm, hf_hbm, out_hbm,
             idsu_v, idsv_v, ubuf_v, vbuf_v, out_v, sems):
    wid = lax.axis_index("s") * NC + lax.axis_index("c")
    base = wid * BPW

    pltpu.sync_copy(uidx_hbm.at[pl.ds(base, BPW)], idsu_v)
    pltpu.sync_copy(vidx_hbm.at[pl.ds(base, BPW)], idsv_v)

    handles = []
    for j in range(NCH):
        hs = []
        for k in range(K):
            hs.append(pltpu.async_copy(
                wf_hbm.at[pl.ds(k * PW, PW)]
                .at[idsu_v.at[pl.ds(j * CHUNK, CHUNK)]],
                ubuf_v.at[j, k], sems.at[j]))
            hs.append(pltpu.async_copy(
                hf_hbm.at[pl.ds(k * PH, PH)]
                .at[idsv_v.at[pl.ds(j * CHUNK, CHUNK)]],
                vbuf_v.at[j, k], sems.at[j]))
        handles.append(hs)

    for j in range(NCH):
        for h in handles[j]:
            h.wait()

        def blk_body(b, carry, j=j):
            acc = jnp.zeros((L,), jnp.float32)
            for k in range(K):
                u = ubuf_v[j, k, pl.ds(b * L, L)]
                v = vbuf_v[j, k, pl.ds(b * L, L)]
                acc = acc + u * v
            out_v[pl.ds(j * CHUNK + b * L, L)] = acc
            return carry

        lax.fori_loop(0, NBLK, blk_body, 0)

    pltpu.sync_copy(out_v, out_hbm.at[pl.ds(base, BPW)])


_MESH = dict(core_axis_name="c", subcore_axis_name="s")


@jax.jit
def kernel(x, W, H):
    u_idx = x[:, 0].astype(jnp.int32)
    v_idx = x[:, 1].astype(jnp.int32)

    fmt = functools.partial(
        pl.kernel,
        mesh=plsc.VectorSubcoreMesh(**_MESH),
        out_type=(
            jax.ShapeDtypeStruct((K * PW,), jnp.float32),
            jax.ShapeDtypeStruct((K * PH,), jnp.float32),
        ),
        compiler_params=pltpu.CompilerParams(
            needs_layout_passes=False,
            use_tc_tiling_on_sc=True,
            disable_bounds_checks=True,
        ),
        scratch_types=[
            pltpu.VMEM((2, K, PIDS), jnp.float32),
            pltpu.VMEM((K, TAIL_W), jnp.float32),
            pltpu.VMEM((K, TAIL_H), jnp.float32),
            pltpu.SemaphoreType.DMA((2,)),
        ],
    )(_fmt_body)
    wt = W.T
    ht = H.T
    w_flat, h_flat = fmt(wt, ht, wt[:, FULL_W * 128:], ht[:, FULL_H * 128:])

    mf = functools.partial(
        pl.kernel,
        mesh=plsc.VectorSubcoreMesh(**_MESH),
        out_type=jax.ShapeDtypeStruct((BATCH,), jnp.float32),
        compiler_params=pltpu.CompilerParams(
            needs_layout_passes=False,
            use_tc_tiling_on_sc=False,
            disable_bounds_checks=True,
        ),
        scratch_types=[
            pltpu.VMEM((BPW,), jnp.int32),
            pltpu.VMEM((BPW,), jnp.int32),
            pltpu.VMEM((NCH, K, CHUNK), jnp.float32),
            pltpu.VMEM((NCH, K, CHUNK), jnp.float32),
            pltpu.VMEM((BPW,), jnp.float32),
            pltpu.SemaphoreType.DMA((NCH,)),
        ],
    )(_mf_body)
    return mf(u_idx, v_idx, w_flat, h_flat)


# speedup vs baseline: 11.7449x; 1.0653x over previous
import functools

import jax
import jax.numpy as jnp
from jax import lax
from jax.experimental import pallas as pl
from jax.experimental.pallas import tpu as pltpu
from jax.experimental.pallas import tpu_sc as plsc

NC = 2
NS = 16
NW = NC * NS
L = 16
BATCH = 16384
K = 16
BPW = BATCH // NW
CHUNK = 128
NCH = BPW // CHUNK
NBLK = CHUNK // L

NU = 100000
NV = 1000000
CW = (NU + 127) // 128
CH = (NV + 127) // 128
PW = CW * 128
PH = CH * 128

FULL_W = NU // 128
FULL_H = NV // 128
TAIL_W = NU - FULL_W * 128
TAIL_H = NV - FULL_H * 128

PCOLS = 24
PIDS = PCOLS * 128
NP_H = (FULL_H + PCOLS - 1) // PCOLS
NP_W = (FULL_W + PCOLS - 1) // PCOLS
ITER_H = (NP_H + NW - 1) // NW
ITER_W = (NP_W + NW - 1) // NW


def _detile_one(table_hbm, flat_hbm, buf_v, rsems, wsems,
                wid, n_pieces, n_full_cols, stride, n_iter):
    last_start = (n_full_cols - PCOLS) * 128

    def piece_start(i):
        return pl.multiple_of(
            jnp.minimum((wid + i * NW) * PIDS, last_start), 128)

    def write_row(b, k, start):
        return pltpu.async_copy(
            buf_v.at[b, k],
            flat_hbm.at[pl.ds(pl.multiple_of(k * stride + start, 128), PIDS)],
            wsems.at[b])

    n_base = n_pieces // NW
    whandles = [None] * n_base
    rhandle = None
    if n_base > 0:
        rhandle = pltpu.async_copy(
            table_hbm.at[:, pl.ds(piece_start(0), PIDS)], buf_v.at[0],
            rsems.at[0])
    for i in range(n_base):
        b = i % 2
        rhandle.wait()
        if i + 1 < n_base:
            if i >= 1:
                for h in whandles[i - 1]:
                    h.wait()
            rhandle = pltpu.async_copy(
                table_hbm.at[:, pl.ds(piece_start(i + 1), PIDS)],
                buf_v.at[1 - b], rsems.at[1 - b])
        start = piece_start(i)
        whandles[i] = [write_row(b, k, start) for k in range(K)]
    for i in range(max(n_base - 2, 0), n_base):
        for h in whandles[i]:
            h.wait()

    if n_pieces % NW:

        @pl.when(wid + n_base * NW < n_pieces)
        def _tail_piece():
            start = piece_start(n_base)
            pltpu.sync_copy(table_hbm.at[:, pl.ds(start, PIDS)], buf_v.at[0])
            for k in range(K):
                pltpu.sync_copy(
                    buf_v.at[0, k],
                    flat_hbm.at[pl.ds(
                        pl.multiple_of(k * stride + start, 128), PIDS)])


def _fmt_body(wt_hbm, ht_hbm, twt_hbm, tht_hbm, wf_hbm, hf_hbm,
              buf_v, tw_v, th_v, rsems, wsems):
    wid = lax.axis_index("s") * NC + lax.axis_index("c")

    @pl.when(wid == 0)
    def _tails():
        pltpu.sync_copy(twt_hbm, tw_v)
        pltpu.sync_copy(tht_hbm, th_v)
        for k in range(K):
            pltpu.sync_copy(
                tw_v.at[k], wf_hbm.at[pl.ds(k * PW + FULL_W * 128, TAIL_W)])
            pltpu.sync_copy(
                th_v.at[k], hf_hbm.at[pl.ds(k * PH + FULL_H * 128, TAIL_H)])

    _detile_one(wt_hbm, wf_hbm, buf_v, rsems, wsems,
                wid, NP_W, FULL_W, PW, ITER_W)
    _detile_one(ht_hbm, hf_hbm, buf_v, rsems, wsems,
                wid, NP_H, FULL_H, PH, ITER_H)


def _mf_body(uidx_hbm, vidx_hbm, wf_hbm, hf_hbm, out_hbm,
             idsu_v, idsv_v, ubuf_v, vbuf_v, out_v, sems):
    wid = lax.axis_index("s") * NC + lax.axis_index("c")
    base = wid * BPW

    pltpu.sync_copy(uidx_hbm.at[pl.ds(base, BPW)], idsu_v)
    pltpu.sync_copy(vidx_hbm.at[pl.ds(base, BPW)], idsv_v)

    handles = []
    for j in range(NCH):
        hs = []
        for k in range(K):
            hs.append(pltpu.async_copy(
                wf_hbm.at[pl.ds(k * PW, PW)]
                .at[idsu_v.at[pl.ds(j * CHUNK, CHUNK)]],
                ubuf_v.at[j, k], sems.at[j]))
            hs.append(pltpu.async_copy(
                hf_hbm.at[pl.ds(k * PH, PH)]
                .at[idsv_v.at[pl.ds(j * CHUNK, CHUNK)]],
                vbuf_v.at[j, k], sems.at[j]))
        handles.append(hs)

    for j in range(NCH):
        for h in handles[j]:
            h.wait()

        def blk_body(b, carry, j=j):
            acc = jnp.zeros((L,), jnp.float32)
            for k in range(K):
                u = ubuf_v[j, k, pl.ds(b * L, L)]
                v = vbuf_v[j, k, pl.ds(b * L, L)]
                acc = acc + u * v
            out_v[pl.ds(j * CHUNK + b * L, L)] = acc
            return carry

        lax.fori_loop(0, NBLK, blk_body, 0)

    pltpu.sync_copy(out_v, out_hbm.at[pl.ds(base, BPW)])


_MESH = dict(core_axis_name="c", subcore_axis_name="s")


@jax.jit
def kernel(x, W, H):
    u_idx = x[:, 0].astype(jnp.int32)
    v_idx = x[:, 1].astype(jnp.int32)

    fmt = functools.partial(
        pl.kernel,
        mesh=plsc.VectorSubcoreMesh(**_MESH),
        out_type=(
            jax.ShapeDtypeStruct((K * PW,), jnp.float32),
            jax.ShapeDtypeStruct((K * PH,), jnp.float32),
        ),
        compiler_params=pltpu.CompilerParams(
            needs_layout_passes=False,
            use_tc_tiling_on_sc=True,
            disable_bounds_checks=True,
        ),
        scratch_types=[
            pltpu.VMEM((2, K, PIDS), jnp.float32),
            pltpu.VMEM((K, TAIL_W), jnp.float32),
            pltpu.VMEM((K, TAIL_H), jnp.float32),
            pltpu.SemaphoreType.DMA((2,)),
            pltpu.SemaphoreType.DMA((2,)),
        ],
    )(_fmt_body)
    wt = W.T
    ht = H.T
    w_flat, h_flat = fmt(wt, ht, wt[:, FULL_W * 128:], ht[:, FULL_H * 128:])

    mf = functools.partial(
        pl.kernel,
        mesh=plsc.VectorSubcoreMesh(**_MESH),
        out_type=jax.ShapeDtypeStruct((BATCH,), jnp.float32),
        compiler_params=pltpu.CompilerParams(
            needs_layout_passes=False,
            use_tc_tiling_on_sc=False,
            disable_bounds_checks=True,
        ),
        scratch_types=[
            pltpu.VMEM((BPW,), jnp.int32),
            pltpu.VMEM((BPW,), jnp.int32),
            pltpu.VMEM((NCH, K, CHUNK), jnp.float32),
            pltpu.VMEM((NCH, K, CHUNK), jnp.float32),
            pltpu.VMEM((BPW,), jnp.float32),
            pltpu.SemaphoreType.DMA((NCH,)),
        ],
    )(_mf_body)
    return mf(u_idx, v_idx, w_flat, h_flat)
